# baseline (speedup 1.0000x reference)
import jax
import jax.numpy as jnp
from jax.experimental import pallas as pl
from jax.experimental.pallas import tpu as pltpu

_BLOCK = 256


def _diag_block_body(x_ref, s_ref, o_ref):
    blk = x_ref[...]
    r = jax.lax.broadcasted_iota(jnp.int32, blk.shape, 0)
    c = jax.lax.broadcasted_iota(jnp.int32, blk.shape, 1)
    o_ref[...] = blk + jnp.where(r == c, s_ref[0], jnp.float32(0.0))


def kernel(x, sigma2):
    n, m = x.shape
    b = _BLOCK if n % _BLOCK == 0 else n
    return pl.pallas_call(
        _diag_block_body,
        grid=(n // b,),
        in_specs=[
            pl.BlockSpec((b, b), lambda i: (i, i)),
            pl.BlockSpec(memory_space=pltpu.SMEM),
        ],
        out_specs=pl.BlockSpec((b, b), lambda i: (i, i)),
        out_shape=jax.ShapeDtypeStruct((n, m), x.dtype),
        input_output_aliases={0: 0},
    )(x, sigma2)


# speedup vs baseline: 1.1225x; 1.1225x over previous
import jax
import jax.numpy as jnp
from jax.experimental import pallas as pl
from jax.experimental.pallas import tpu as pltpu

_BLOCK_ROWS = 256


def _diag_add_body(x_ref, s_ref, o_ref):
    i = pl.program_id(0)
    blk = x_ref[...]
    rows, cols = blk.shape
    r = jax.lax.broadcasted_iota(jnp.int32, (rows, cols), 0)
    c = jax.lax.broadcasted_iota(jnp.int32, (rows, cols), 1)
    mask = c == r + i * rows
    o_ref[...] = blk + jnp.where(mask, s_ref[0], jnp.float32(0.0))


def kernel(x, sigma2):
    n, m = x.shape
    br = _BLOCK_ROWS if n % _BLOCK_ROWS == 0 else n
    grid = (n // br,)
    return pl.pallas_call(
        _diag_add_body,
        grid=grid,
        in_specs=[
            pl.BlockSpec((br, m), lambda i: (i, 0)),
            pl.BlockSpec(memory_space=pltpu.SMEM),
        ],
        out_specs=pl.BlockSpec((br, m), lambda i: (i, 0)),
        out_shape=jax.ShapeDtypeStruct((n, m), x.dtype),
        compiler_params=pltpu.CompilerParams(
            dimension_semantics=("parallel",),
        ),
    )(x, sigma2)
